# baseline (speedup 1.0000x reference)
import functools
import math

import jax
import jax.numpy as jnp
from jax import lax
from jax.experimental import pallas as pl
from jax.experimental.pallas import tpu as pltpu
from jax.experimental.pallas import tpu_sc as plsc

D_MODEL = 1024
SCALE = math.sqrt(D_MODEL)

NUM_CORES = 2
NUM_SUBCORES = 16
LANES = 16
NUM_WORKERS = NUM_CORES * NUM_SUBCORES


def _sc_embed(idx_flat, table):
    b_total = idx_flat.shape[0]
    b_per_w = b_total // NUM_WORKERS
    chunk = 64
    n_chunks = b_per_w // chunk

    mesh = plsc.VectorSubcoreMesh(
        core_axis_name="c",
        subcore_axis_name="s",
        num_cores=NUM_CORES,
        num_subcores=NUM_SUBCORES,
    )

    @functools.partial(
        pl.kernel,
        mesh=mesh,
        out_type=jax.ShapeDtypeStruct((b_total, D_MODEL), jnp.float32),
        scratch_types=[
            pltpu.VMEM((b_per_w,), jnp.int32),
            pltpu.VMEM((chunk, D_MODEL), jnp.float32),
            pltpu.SemaphoreType.DMA,
        ],
    )
    def k(idx_hbm, table_hbm, out_hbm, idx_v, rows_v, sem):
        wid = lax.axis_index("s") * NUM_CORES + lax.axis_index("c")
        base = wid * b_per_w
        pltpu.sync_copy(idx_hbm.at[pl.ds(base, b_per_w)], idx_v)

        def chunk_body(c, _):
            pltpu.async_copy(
                table_hbm.at[idx_v.at[pl.ds(c * chunk, chunk)]], rows_v, sem
            ).wait()

            def row_body(r, _):
                def col_body(j, _):
                    sl = pl.ds(j * LANES, LANES)
                    rows_v[r, sl] = rows_v[r, sl] * SCALE
                    return 0

                return lax.fori_loop(0, D_MODEL // LANES, col_body, 0)

            lax.fori_loop(0, chunk, row_body, 0)
            pltpu.sync_copy(rows_v, out_hbm.at[pl.ds(base + c * chunk, chunk)])
            return 0

        lax.fori_loop(0, n_chunks, chunk_body, 0)

    return k(idx_flat, table)


def kernel(x, table):
    idx_flat = x.reshape(-1).astype(jnp.int32)
    out = _sc_embed(idx_flat, table)
    return out.reshape(x.shape + (D_MODEL,))


# speedup vs baseline: 2.8469x; 2.8469x over previous
import functools
import math

import jax
import jax.numpy as jnp
from jax import lax
from jax.experimental import pallas as pl
from jax.experimental.pallas import tpu as pltpu
from jax.experimental.pallas import tpu_sc as plsc

D_MODEL = 1024
SCALE = math.sqrt(D_MODEL)

NUM_CORES = 2
NUM_SUBCORES = 16
LANES = 16
NUM_WORKERS = NUM_CORES * NUM_SUBCORES

CHUNK = 16


def _sc_embed(idx_flat, table):
    b_total = idx_flat.shape[0]
    b_per_w = b_total // NUM_WORKERS
    n_chunks = b_per_w // CHUNK
    n_vec = CHUNK * D_MODEL // LANES
    col_mask = D_MODEL // LANES - 1

    mesh = plsc.VectorSubcoreMesh(
        core_axis_name="c",
        subcore_axis_name="s",
        num_cores=NUM_CORES,
        num_subcores=NUM_SUBCORES,
    )

    @functools.partial(
        pl.kernel,
        mesh=mesh,
        out_type=jax.ShapeDtypeStruct((b_total, D_MODEL), jnp.float32),
        scratch_types=[
            pltpu.VMEM((b_per_w,), jnp.int32),
            pltpu.VMEM((CHUNK, D_MODEL), jnp.float32),
            pltpu.VMEM((CHUNK, D_MODEL), jnp.float32),
            pltpu.VMEM((CHUNK, D_MODEL), jnp.float32),
            pltpu.VMEM((CHUNK, D_MODEL), jnp.float32),
            pltpu.SemaphoreType.DMA,
            pltpu.SemaphoreType.DMA,
            pltpu.SemaphoreType.DMA,
            pltpu.SemaphoreType.DMA,
        ],
    )
    def k(idx_hbm, table_hbm, out_hbm, idx_v, in0, in1, st0, st1,
          gs0, gs1, ws0, ws1):
        ins = (in0, in1)
        outs = (st0, st1)
        gsem = (gs0, gs1)
        wsem = (ws0, ws1)

        wid = lax.axis_index("s") * NUM_CORES + lax.axis_index("c")
        base = wid * b_per_w
        pltpu.sync_copy(idx_hbm.at[pl.ds(base, b_per_w)], idx_v)

        def gather_start(g, b):
            pltpu.async_copy(
                table_hbm.at[idx_v.at[pl.ds(g * CHUNK, CHUNK)]], ins[b], gsem[b]
            )

        def gather_wait(g, b):
            pltpu.make_async_copy(
                table_hbm.at[idx_v.at[pl.ds(g * CHUNK, CHUNK)]], ins[b], gsem[b]
            ).wait()

        def write_start(g, b):
            pltpu.async_copy(
                outs[b], out_hbm.at[pl.ds(base + g * CHUNK, CHUNK)], wsem[b]
            )

        def write_wait(g, b):
            pltpu.make_async_copy(
                outs[b], out_hbm.at[pl.ds(base + g * CHUNK, CHUNK)], wsem[b]
            ).wait()

        def scale(b):
            src = ins[b]
            dst = outs[b]

            @plsc.parallel_loop(0, n_vec, unroll=8)
            def _(i):
                r = lax.shift_right_logical(i, 6)
                sl = pl.ds((i & col_mask) * LANES, LANES)
                dst[r, sl] = src[r, sl] * SCALE

        gather_start(0, 0)
        gather_start(1, 1)

        for b in range(2):
            gather_wait(b, b)
            scale(b)
            write_start(b, b)
            gather_start(b + 2, b)

        def body(g2, _):
            for b in range(2):
                g = 2 * g2 + b
                gather_wait(g, b)
                write_wait(g - 2, b)
                scale(b)
                write_start(g, b)
                gather_start(g + 2, b)
            return 0

        lax.fori_loop(1, n_chunks // 2 - 1, body, 0)

        for b in range(2):
            g = n_chunks - 2 + b
            gather_wait(g, b)
            write_wait(g - 2, b)
            scale(b)
            write_start(g, b)
        for b in range(2):
            write_wait(n_chunks - 2 + b, b)

    return k(idx_flat, table)


def kernel(x, table):
    idx_flat = x.reshape(-1).astype(jnp.int32)
    out = _sc_embed(idx_flat, table)
    return out.reshape(x.shape + (D_MODEL,))


# speedup vs baseline: 2.8501x; 1.0011x over previous
import functools
import math

import jax
import jax.numpy as jnp
from jax import lax
from jax.experimental import pallas as pl
from jax.experimental.pallas import tpu as pltpu
from jax.experimental.pallas import tpu_sc as plsc

D_MODEL = 1024
SCALE = math.sqrt(D_MODEL)

NUM_CORES = 2
NUM_SUBCORES = 16
LANES = 16
NUM_WORKERS = NUM_CORES * NUM_SUBCORES

CHUNK = 16


def _sc_embed(idx_flat, table):
    b_total = idx_flat.shape[0]
    b_per_w = b_total // NUM_WORKERS
    n_chunks = b_per_w // CHUNK
    n_vec = CHUNK * D_MODEL // LANES
    col_mask = D_MODEL // LANES - 1

    mesh = plsc.VectorSubcoreMesh(
        core_axis_name="c",
        subcore_axis_name="s",
        num_cores=NUM_CORES,
        num_subcores=NUM_SUBCORES,
    )

    @functools.partial(
        pl.kernel,
        mesh=mesh,
        out_type=jax.ShapeDtypeStruct((b_total, D_MODEL), jnp.float32),
        scratch_types=[
            pltpu.VMEM((b_per_w,), jnp.int32),
            pltpu.VMEM((CHUNK, D_MODEL), jnp.float32),
            pltpu.VMEM((CHUNK, D_MODEL), jnp.float32),
            pltpu.VMEM((CHUNK, D_MODEL), jnp.float32),
            pltpu.VMEM((CHUNK, D_MODEL), jnp.float32),
            pltpu.SemaphoreType.DMA,
            pltpu.SemaphoreType.DMA,
            pltpu.SemaphoreType.DMA,
            pltpu.SemaphoreType.DMA,
        ],
    )
    def k(idx_hbm, table_hbm, out_hbm, idx_v, in0, in1, st0, st1,
          gs0, gs1, ws0, ws1):
        ins = (in0, in1)
        outs = (st0, st1)
        gsem = (gs0, gs1)
        wsem = (ws0, ws1)

        wid = lax.axis_index("s") * NUM_CORES + lax.axis_index("c")
        base = wid * b_per_w
        pltpu.sync_copy(idx_hbm.at[pl.ds(base, b_per_w)], idx_v)

        def gather_start(g, b):
            pltpu.async_copy(
                table_hbm.at[idx_v.at[pl.ds(g * CHUNK, CHUNK)]], ins[b], gsem[b]
            )

        def gather_wait(g, b):
            pltpu.make_async_copy(
                table_hbm.at[idx_v.at[pl.ds(g * CHUNK, CHUNK)]], ins[b], gsem[b]
            ).wait()

        def write_start(g, b):
            pltpu.async_copy(
                outs[b], out_hbm.at[pl.ds(base + g * CHUNK, CHUNK)], wsem[b]
            )

        def write_wait(g, b):
            pltpu.make_async_copy(
                outs[b], out_hbm.at[pl.ds(base + g * CHUNK, CHUNK)], wsem[b]
            ).wait()

        def scale(b):
            src = ins[b]
            dst = outs[b]

            @plsc.parallel_loop(0, n_vec, unroll=16)
            def _(i):
                r = lax.shift_right_logical(i, 6)
                sl = pl.ds((i & col_mask) * LANES, LANES)
                dst[r, sl] = src[r, sl] * SCALE

        gather_start(0, 0)
        gather_start(1, 1)

        for b in range(2):
            gather_wait(b, b)
            scale(b)
            write_start(b, b)
            gather_start(b + 2, b)

        def body(g2, _):
            for b in range(2):
                g = 2 * g2 + b
                gather_wait(g, b)
                write_wait(g - 2, b)
                scale(b)
                write_start(g, b)
                gather_start(g + 2, b)
            return 0

        lax.fori_loop(1, n_chunks // 2 - 1, body, 0)

        for b in range(2):
            g = n_chunks - 2 + b
            gather_wait(g, b)
            write_wait(g - 2, b)
            scale(b)
            write_start(g, b)
        for b in range(2):
            write_wait(n_chunks - 2 + b, b)

    return k(idx_flat, table)


def kernel(x, table):
    idx_flat = x.reshape(-1).astype(jnp.int32)
    out = _sc_embed(idx_flat, table)
    return out.reshape(x.shape + (D_MODEL,))
